# baseline (speedup 1.0000x reference)
import functools

import jax
import jax.numpy as jnp
from jax import lax
from jax.experimental import pallas as pl
from jax.experimental.pallas import tpu as pltpu
from jax.experimental.pallas import tpu_sc as plsc

B = 4096
S = 200
D = 64
OUT = 64
NC = 2
NS = 16
NW = NC * NS
BPW = B // NW
CH0 = 104
CH1 = S - CH0
NLANE = 16
NJ = D // NLANE
VP = 500_000
BLK_STARTS = tuple(k * NLANE for k in range(S // NLANE)) + (S - NLANE,)


def _pooled_body(tok_hbm, table_hbm, out_hbm,
                 tok_v, pair0_v, pair1_v, rows_v, pooled_v, sem0, sem1):
    wid = lax.axis_index("s") * NC + lax.axis_index("c")
    base = wid * BPW
    pltpu.sync_copy(tok_hbm.at[pl.ds(base, BPW)], tok_v)

    sems = (sem0, sem1)
    pairs = (pair0_v, pair1_v)

    def issue(i, nb):
        pv = pairs[nb]
        for st in BLK_STARTS:
            toki = plsc.bitcast(tok_v[i, pl.ds(st, NLANE)], jnp.int32)
            pv[pl.ds(st, NLANE)] = toki >> 1
        pltpu.async_copy(
            table_hbm.at[pv.at[pl.ds(0, CH0)]],
            rows_v.at[nb, pl.ds(0, CH0)], sems[nb])
        pltpu.async_copy(
            table_hbm.at[pv.at[pl.ds(CH0, CH1)]],
            rows_v.at[nb, pl.ds(CH0, CH1)], sems[nb])

    def drain(nb):
        pv = pairs[nb]
        pltpu.make_async_copy(
            table_hbm.at[pv.at[pl.ds(0, CH0)]],
            rows_v.at[nb, pl.ds(0, CH0)], sems[nb]).wait()
        pltpu.make_async_copy(
            table_hbm.at[pv.at[pl.ds(CH0, CH1)]],
            rows_v.at[nb, pl.ds(CH0, CH1)], sems[nb]).wait()

    issue(0, 0)
    issue(1, 1)

    def group_body(g, carry):
        for nb in range(2):
            i = g * 2 + nb
            drain(nb)

            def blk_body(t, accs):
                tokv = plsc.bitcast(tok_v[i, pl.ds(t * NLANE, NLANE)],
                                    jnp.int32)
                for u in range(NLANE):
                    s_ = t * NLANE + u
                    off = (tokv[u] & 1) * D
                    accs = tuple(
                        accs[j] + rows_v[nb, s_, pl.ds(off + j * NLANE, NLANE)]
                        for j in range(NJ)
                    )
                return accs

            accs = lax.fori_loop(
                0, S // NLANE, blk_body,
                tuple(jnp.zeros((NLANE,), jnp.float32) for _ in range(NJ)),
            )
            tokv_t = plsc.bitcast(tok_v[i, pl.ds(S - NLANE, NLANE)], jnp.int32)
            for u in range(NLANE - S % NLANE, NLANE):
                s_ = S - NLANE + u
                off = (tokv_t[u] & 1) * D
                accs = tuple(
                    accs[j] + rows_v[nb, s_, pl.ds(off + j * NLANE, NLANE)]
                    for j in range(NJ)
                )

            @pl.when(i + 2 < BPW)
            def _():
                issue(i + 2, nb)

            for j in range(NJ):
                pooled_v[i, pl.ds(j * NLANE, NLANE)] = accs[j] * (1.0 / S)
        return carry

    lax.fori_loop(0, BPW // 2, group_body, 0)
    pltpu.sync_copy(pooled_v, out_hbm.at[pl.ds(base, BPW)])


_pooled = functools.partial(
    pl.kernel,
    out_type=jax.ShapeDtypeStruct((B, D), jnp.float32),
    mesh=plsc.VectorSubcoreMesh(core_axis_name="c", subcore_axis_name="s"),
    scratch_types=[
        pltpu.VMEM((BPW, S), jnp.float32),
        pltpu.VMEM((S,), jnp.int32),
        pltpu.VMEM((S,), jnp.int32),
        pltpu.VMEM((2, S, 2 * D), jnp.float32),
        pltpu.VMEM((BPW, D), jnp.float32),
        pltpu.SemaphoreType.DMA,
        pltpu.SemaphoreType.DMA,
    ],
    compiler_params=pltpu.CompilerParams(
        use_tc_tiling_on_sc=True, needs_layout_passes=False),
)(_pooled_body)


def _linear_body(x_ref, w_ref, b_ref, o_ref):
    o_ref[...] = (
        lax.dot_general(
            x_ref[...], w_ref[...],
            (((1,), (1,)), ((), ())),
            preferred_element_type=jnp.float32,
        )
        + b_ref[...]
    )


_linear = pl.pallas_call(
    _linear_body,
    out_shape=jax.ShapeDtypeStruct((B, OUT), jnp.float32),
    grid=(8,),
    in_specs=[
        pl.BlockSpec((B // 8, D), lambda i: (i, 0)),
        pl.BlockSpec((OUT, D), lambda i: (0, 0)),
        pl.BlockSpec((1, OUT), lambda i: (0, 0)),
    ],
    out_specs=pl.BlockSpec((B // 8, OUT), lambda i: (i, 0)),
)


def kernel(token_ids, table, W, b):
    tokf = lax.bitcast_convert_type(token_ids.astype(jnp.int32), jnp.float32)
    tab2 = table.reshape(VP, 2 * D)
    pooled = _pooled(tokf, tab2)
    return _linear(pooled, W, b.reshape(1, OUT))
